# baseline (speedup 1.0000x reference)
import dataclasses

import jax
import jax.numpy as jnp
from jax import lax
from jax.experimental import pallas as pl
from jax.experimental.pallas import tpu as pltpu
from jax.experimental.pallas import tpu_sc as plsc

_S = 512
_C = 96
_N = 2 * _C
_HALF = 10
_INV_BW = 1.0 / 21.0
_G = 8
_GRAN = 32
_NROWS = _N * _S * _S // _GRAN
_PPW = 2
_NSC = 32 * _PPW


def _sc_start(g):
    return min(max(((16 * g - _HALF) // 128) * 128, 0), _S - 256)


def _sc_band_kernel(x_hbm, band_hbm, buf_v, acc_v, sem):
    wid = lax.axis_index("s") * 2 + lax.axis_index("c")
    iot = lax.iota(jnp.int32, 16)
    zeros16 = jnp.zeros((16,), jnp.float32)

    @pl.loop(0, _PPW)
    def _(t):
        p = wid * _PPW + t

        @pl.loop(0, 256)
        def _(i):
            q = 16 * i + iot
            plsc.store_scatter(acc_v, [jnp.right_shift(q, 9),
                                       jnp.bitwise_and(q, 511)], zeros16)

        for rnd in range(2):
            copies = [
                pltpu.async_copy(
                    x_hbm.at[p, pl.ds(16 * (16 * rnd + s), 16),
                             pl.ds(_sc_start(16 * rnd + s), 256)],
                    buf_v.at[s], sem)
                for s in range(16)
            ]
            for cp in copies:
                cp.wait()

            @pl.loop(0, 16)
            def _(s):
                g = 16 * rnd + s
                sv = iot * 0 + s
                base = g * 16 - _HALF
                sC = jnp.minimum(
                    jnp.maximum(jnp.left_shift(jnp.right_shift(base, 7), 7), 0),
                    _S - 256)
                for j in range(21):
                    colv = iot + (base + j)
                    m = (colv >= 0) & (colv < _S)
                    cidx = jnp.minimum(jnp.maximum(colv - sC, 0), 255)
                    v = plsc.load_gather(buf_v, [sv, iot, cidx])
                    q = colv + 16
                    plsc.addupdate_scatter(acc_v, [jnp.right_shift(q, 9),
                                                   jnp.bitwise_and(q, 511)],
                                           jnp.where(m, v, 0.0))

        pltpu.sync_copy(acc_v, band_hbm.at[p])


def _tc_band_kernel(x_ref, band_ref):
    xb = x_ref[...]
    r = jax.lax.broadcasted_iota(jnp.int32, (1, _S, _S), 1)
    c = jax.lax.broadcasted_iota(jnp.int32, (1, _S, _S), 2)
    d = c - r
    in_band = (d >= -_HALF) & (d <= _HALF)
    band_ref[:, 0, :] = jnp.sum(jnp.where(in_band, xb, 0.0), axis=1)


def _attn_kernel(band_ref, cw_ref, pw_ref, pb_ref, out_ref):
    band = band_ref[...]
    cw = cw_ref[...]
    bp = jnp.pad(band, ((0, 0), (3, 3)))
    attn = cw[:, 0:1] * bp[:, 0:_S]
    for k in range(1, 7):
        attn = attn + cw[:, k:k + 1] * bp[:, k:k + _S]
    pw = pw_ref[...]
    attn = jnp.dot(pw, attn, preferred_element_type=jnp.float32) + pb_ref[...]
    m = jnp.max(attn, axis=1, keepdims=True)
    e = jnp.exp(attn - m)
    out_ref[...] = e / jnp.sum(e, axis=1, keepdims=True)


def _copy_sub_kernel(x_ref, attn_ref, y_ref):
    xb = x_ref[...]
    at = attn_ref[...]
    r = jax.lax.broadcasted_iota(jnp.int32, (1, _S, _S), 1)
    c = jax.lax.broadcasted_iota(jnp.int32, (1, _S, _S), 2)
    y_ref[...] = jnp.where(r == c, at * xb, xb)


def _copy_sub_kernel2(y_in_ref, x_ref, attn_ref, y_ref):
    del y_in_ref
    _copy_sub_kernel(x_ref, attn_ref, y_ref)


def _attn_call(band, cw, pw, pb):
    return pl.pallas_call(
        _attn_kernel,
        out_shape=jax.ShapeDtypeStruct((_C, _S), jnp.float32),
    )(band, cw, pw, pb)


def kernel(x, conv_w, point_w, point_b):
    b, c, h, w = x.shape
    x3 = x.reshape(_N, _S, _S)

    mesh = plsc.VectorSubcoreMesh(core_axis_name="c", subcore_axis_name="s")
    cp = pltpu.CompilerParams()
    if "needs_layout_passes" in pltpu.CompilerParams.__dataclass_fields__:
        cp = dataclasses.replace(cp, needs_layout_passes=False,
                                 use_tc_tiling_on_sc=True)
    sc_band = pl.kernel(
        _sc_band_kernel,
        out_type=jax.ShapeDtypeStruct((_NSC, 8, _S), jnp.float32),
        mesh=mesh,
        scratch_types=[
            pltpu.VMEM((16, 16, 256), jnp.float32),
            pltpu.VMEM((8, _S), jnp.float32),
            pltpu.SemaphoreType.DMA,
        ],
        compiler_params=cp,
    )
    braw = sc_band(x3)
    band_sc = jnp.concatenate([braw[:, 0, 16:], braw[:, 1, :16]], axis=-1)

    band_tc = pl.pallas_call(
        _tc_band_kernel,
        grid=((_N - _NSC) // _G,),
        in_specs=[pl.BlockSpec((_G, _S, _S),
                               lambda n: (n + _NSC // _G, 0, 0))],
        out_specs=pl.BlockSpec((_G, 1, _S), lambda n: (n, 0, 0)),
        out_shape=jax.ShapeDtypeStruct((_N - _NSC, 1, _S), jnp.float32),
    )(x3).reshape(_N - _NSC, _S)

    band0 = jnp.concatenate([band_sc, band_tc[:_C - _NSC]], axis=0)
    band1 = band_tc[_C - _NSC:]

    cw = conv_w.reshape(_C, 7) * _INV_BW
    pw = point_w.reshape(_C, _C)
    pb = point_b.reshape(_C, 1)

    attn1 = _attn_call(band1, cw, pw, pb).reshape(_C, 1, _S)
    out_half = pl.pallas_call(
        _copy_sub_kernel,
        grid=(_C // _G,),
        in_specs=[
            pl.BlockSpec((_G, _S, _S), lambda n: (n + _C // _G, 0, 0)),
            pl.BlockSpec((_G, 1, _S), lambda n: (n, 0, 0)),
        ],
        out_specs=pl.BlockSpec((_G, _S, _S), lambda n: (n + _C // _G, 0, 0)),
        out_shape=jax.ShapeDtypeStruct((_N, _S, _S), jnp.float32),
    )(x3, attn1)

    attn0 = _attn_call(band0, cw, pw, pb).reshape(_C, 1, _S)
    out = pl.pallas_call(
        _copy_sub_kernel2,
        grid=(_C // _G,),
        in_specs=[
            pl.BlockSpec(memory_space=pl.ANY),
            pl.BlockSpec((_G, _S, _S), lambda n: (n, 0, 0)),
            pl.BlockSpec((_G, 1, _S), lambda n: (n, 0, 0)),
        ],
        out_specs=pl.BlockSpec((_G, _S, _S), lambda n: (n, 0, 0)),
        out_shape=jax.ShapeDtypeStruct((_N, _S, _S), jnp.float32),
        input_output_aliases={0: 0},
    )(out_half, x3, attn0)

    return out.reshape(b, c, h, w)


# speedup vs baseline: 1.0412x; 1.0412x over previous
import dataclasses

import jax
import jax.numpy as jnp
from jax import lax
from jax.experimental import pallas as pl
from jax.experimental.pallas import tpu as pltpu
from jax.experimental.pallas import tpu_sc as plsc

_S = 512
_C = 96
_N = 2 * _C
_HALF = 10
_INV_BW = 1.0 / 21.0
_G = 8
_GRAN = 32
_NROWS = _N * _S * _S // _GRAN
_PPW = 3
_NSC = 32 * _PPW


def _sc_start(g):
    return min(max(((16 * g - _HALF) // 128) * 128, 0), _S - 256)


def _sc_band_kernel(x_hbm, band_hbm, buf_v, acc_v, sem):
    wid = lax.axis_index("s") * 2 + lax.axis_index("c")
    iot = lax.iota(jnp.int32, 16)
    zeros16 = jnp.zeros((16,), jnp.float32)

    @pl.loop(0, _PPW)
    def _(t):
        p = wid * _PPW + t

        @pl.loop(0, 256)
        def _(i):
            q = 16 * i + iot
            plsc.store_scatter(acc_v, [jnp.right_shift(q, 9),
                                       jnp.bitwise_and(q, 511)], zeros16)

        for rnd in range(2):
            copies = [
                pltpu.async_copy(
                    x_hbm.at[p, pl.ds(16 * (16 * rnd + s), 16),
                             pl.ds(_sc_start(16 * rnd + s), 256)],
                    buf_v.at[s], sem)
                for s in range(16)
            ]
            for cp in copies:
                cp.wait()

            @pl.loop(0, 16)
            def _(s):
                g = 16 * rnd + s
                sv = iot * 0 + s
                base = g * 16 - _HALF
                sC = jnp.minimum(
                    jnp.maximum(jnp.left_shift(jnp.right_shift(base, 7), 7), 0),
                    _S - 256)
                for j in range(21):
                    colv = iot + (base + j)
                    m = (colv >= 0) & (colv < _S)
                    cidx = jnp.minimum(jnp.maximum(colv - sC, 0), 255)
                    v = plsc.load_gather(buf_v, [sv, iot, cidx])
                    q = colv + 16
                    plsc.addupdate_scatter(acc_v, [jnp.right_shift(q, 9),
                                                   jnp.bitwise_and(q, 511)],
                                           jnp.where(m, v, 0.0))

        pltpu.sync_copy(acc_v, band_hbm.at[p])


def _tc_band_kernel(x_ref, band_ref):
    xb = x_ref[...]
    r = jax.lax.broadcasted_iota(jnp.int32, (1, _S, _S), 1)
    c = jax.lax.broadcasted_iota(jnp.int32, (1, _S, _S), 2)
    d = c - r
    in_band = (d >= -_HALF) & (d <= _HALF)
    band_ref[:, 0, :] = jnp.sum(jnp.where(in_band, xb, 0.0), axis=1)


def _attn_kernel(band_ref, cw_ref, pw_ref, pb_ref, out_ref):
    band = band_ref[...]
    cw = cw_ref[...]
    bp = jnp.pad(band, ((0, 0), (3, 3)))
    attn = cw[:, 0:1] * bp[:, 0:_S]
    for k in range(1, 7):
        attn = attn + cw[:, k:k + 1] * bp[:, k:k + _S]
    pw = pw_ref[...]
    attn = jnp.dot(pw, attn, preferred_element_type=jnp.float32) + pb_ref[...]
    m = jnp.max(attn, axis=1, keepdims=True)
    e = jnp.exp(attn - m)
    out_ref[...] = e / jnp.sum(e, axis=1, keepdims=True)


def _copy_sub_kernel(x_ref, attn_ref, y_ref):
    xb = x_ref[...]
    at = attn_ref[...]
    r = jax.lax.broadcasted_iota(jnp.int32, (1, _S, _S), 1)
    c = jax.lax.broadcasted_iota(jnp.int32, (1, _S, _S), 2)
    y_ref[...] = jnp.where(r == c, at * xb, xb)


def _copy_sub_kernel2(y_in_ref, x_ref, attn_ref, y_ref):
    del y_in_ref
    _copy_sub_kernel(x_ref, attn_ref, y_ref)


def _attn_call(band, cw, pw, pb):
    return pl.pallas_call(
        _attn_kernel,
        out_shape=jax.ShapeDtypeStruct((_C, _S), jnp.float32),
    )(band, cw, pw, pb)


def kernel(x, conv_w, point_w, point_b):
    b, c, h, w = x.shape
    x3 = x.reshape(_N, _S, _S)

    mesh = plsc.VectorSubcoreMesh(core_axis_name="c", subcore_axis_name="s")
    cp = pltpu.CompilerParams()
    if "needs_layout_passes" in pltpu.CompilerParams.__dataclass_fields__:
        cp = dataclasses.replace(cp, needs_layout_passes=False,
                                 use_tc_tiling_on_sc=True)
    sc_band = pl.kernel(
        _sc_band_kernel,
        out_type=jax.ShapeDtypeStruct((_NSC, 8, _S), jnp.float32),
        mesh=mesh,
        scratch_types=[
            pltpu.VMEM((16, 16, 256), jnp.float32),
            pltpu.VMEM((8, _S), jnp.float32),
            pltpu.SemaphoreType.DMA,
        ],
        compiler_params=cp,
    )
    braw = sc_band(x3)
    band_sc = jnp.concatenate([braw[:, 0, 16:], braw[:, 1, :16]], axis=-1)

    band_tc = pl.pallas_call(
        _tc_band_kernel,
        grid=((_N - _NSC) // _G,),
        in_specs=[pl.BlockSpec((_G, _S, _S),
                               lambda n: (n + _NSC // _G, 0, 0))],
        out_specs=pl.BlockSpec((_G, 1, _S), lambda n: (n, 0, 0)),
        out_shape=jax.ShapeDtypeStruct((_N - _NSC, 1, _S), jnp.float32),
    )(x3).reshape(_N - _NSC, _S)

    band0 = jnp.concatenate([band_sc, band_tc[:_C - _NSC]], axis=0)
    band1 = band_tc[_C - _NSC:]

    cw = conv_w.reshape(_C, 7) * _INV_BW
    pw = point_w.reshape(_C, _C)
    pb = point_b.reshape(_C, 1)

    attn1 = _attn_call(band1, cw, pw, pb).reshape(_C, 1, _S)
    out_half = pl.pallas_call(
        _copy_sub_kernel,
        grid=(_C // _G,),
        in_specs=[
            pl.BlockSpec((_G, _S, _S), lambda n: (n + _C // _G, 0, 0)),
            pl.BlockSpec((_G, 1, _S), lambda n: (n, 0, 0)),
        ],
        out_specs=pl.BlockSpec((_G, _S, _S), lambda n: (n + _C // _G, 0, 0)),
        out_shape=jax.ShapeDtypeStruct((_N, _S, _S), jnp.float32),
    )(x3, attn1)

    attn0 = _attn_call(band0, cw, pw, pb).reshape(_C, 1, _S)
    out = pl.pallas_call(
        _copy_sub_kernel2,
        grid=(_C // _G,),
        in_specs=[
            pl.BlockSpec(memory_space=pl.ANY),
            pl.BlockSpec((_G, _S, _S), lambda n: (n, 0, 0)),
            pl.BlockSpec((_G, 1, _S), lambda n: (n, 0, 0)),
        ],
        out_specs=pl.BlockSpec((_G, _S, _S), lambda n: (n, 0, 0)),
        out_shape=jax.ShapeDtypeStruct((_N, _S, _S), jnp.float32),
        input_output_aliases={0: 0},
    )(out_half, x3, attn0)

    return out.reshape(b, c, h, w)
